# baseline (speedup 1.0000x reference)
import functools

import jax
import jax.numpy as jnp
from jax import lax
from jax.experimental import pallas as pl
from jax.experimental.pallas import tpu as pltpu
from jax.experimental.pallas import tpu_sc as plsc

N = 16384
EMBED = 64
NUM_CORES = 2
NUM_SUBCORES = 16
NW = NUM_CORES * NUM_SUBCORES
BPW = N // NW
CHUNK = 128
NCHUNK = BPW // CHUNK

_mesh = plsc.VectorSubcoreMesh(core_axis_name="c", subcore_axis_name="s")


@functools.partial(
    pl.kernel,
    mesh=_mesh,
    out_type=jax.ShapeDtypeStruct((N, EMBED), jnp.float32),
    scratch_types=[
        pltpu.VMEM((BPW,), jnp.int32),
        pltpu.VMEM((BPW, EMBED), jnp.float32),
        pltpu.SemaphoreType.DMA,
    ],
    compiler_params=pltpu.CompilerParams(use_tc_tiling_on_sc=False),
)
def _gather_kernel(idx_hbm, table_hbm, out_hbm, idx_v, rows_v, sem):
    wid = lax.axis_index("s") * NUM_CORES + lax.axis_index("c")
    base = wid * BPW
    pltpu.sync_copy(idx_hbm.at[pl.ds(base, BPW)], idx_v)
    copies = []
    for j in range(NCHUNK):
        copies.append(
            pltpu.async_copy(
                table_hbm.at[idx_v.at[pl.ds(j * CHUNK, CHUNK)]],
                rows_v.at[pl.ds(j * CHUNK, CHUNK)],
                sem,
            )
        )
    for c in copies:
        c.wait()
    pltpu.sync_copy(rows_v, out_hbm.at[pl.ds(base, BPW)])


def kernel(data, ivectors):
    return _gather_kernel(data.astype(jnp.int32), ivectors)


# speedup vs baseline: 1.0035x; 1.0035x over previous
import functools

import jax
import jax.numpy as jnp
from jax import lax
from jax.experimental import pallas as pl
from jax.experimental.pallas import tpu as pltpu
from jax.experimental.pallas import tpu_sc as plsc

N = 16384
EMBED = 64
NUM_CORES = 2
NUM_SUBCORES = 16
NW = NUM_CORES * NUM_SUBCORES
BPW = N // NW
LANES = 16
NGROUP = BPW // LANES

_mesh = plsc.VectorSubcoreMesh(core_axis_name="c", subcore_axis_name="s")


@functools.partial(
    pl.kernel,
    mesh=_mesh,
    out_type=jax.ShapeDtypeStruct((N * EMBED,), jnp.float32),
    scratch_types=[
        pltpu.VMEM((BPW,), jnp.int32),
        pltpu.VMEM((BPW * EMBED,), jnp.float32),
        pltpu.SemaphoreType.DMA,
    ],
)
def _gather_kernel(idx_hbm, table_hbm, out_hbm, idx_v, rows_v, sem):
    wid = lax.axis_index("s") * NUM_CORES + lax.axis_index("c")
    base = wid * BPW
    pltpu.sync_copy(idx_hbm.at[pl.ds(base, BPW)], idx_v)

    def group(g, _):
        vec = idx_v[pl.ds(g * LANES, LANES)] * EMBED
        for k in range(LANES):
            pltpu.async_copy(
                table_hbm.at[pl.ds(pl.multiple_of(vec[k], EMBED), EMBED)],
                rows_v.at[pl.ds((g * LANES + k) * EMBED, EMBED)],
                sem,
            )
        return 0

    lax.fori_loop(0, NGROUP, group, 0)
    pltpu.make_async_copy(
        table_hbm.at[pl.ds(0, BPW * EMBED)], rows_v, sem
    ).wait()
    pltpu.sync_copy(rows_v, out_hbm.at[pl.ds(base * EMBED, BPW * EMBED)])


def kernel(data, ivectors):
    out = _gather_kernel(
        data.astype(jnp.int32), ivectors.reshape(ivectors.size)
    )
    return out.reshape(N, EMBED)


# speedup vs baseline: 2.5529x; 2.5439x over previous
import functools

import jax
import jax.numpy as jnp
from jax import lax
from jax.experimental import pallas as pl
from jax.experimental.pallas import tpu as pltpu
from jax.experimental.pallas import tpu_sc as plsc

N = 16384
EMBED = 64
GRP = 8
VOCAB = 1000000
NUM_CORES = 2
NUM_SUBCORES = 16
NW = NUM_CORES * NUM_SUBCORES
BPW = N // NW
LANES = 16
NGROUP = BPW // LANES

_mesh = plsc.VectorSubcoreMesh(core_axis_name="c", subcore_axis_name="s")


@functools.partial(
    pl.kernel,
    mesh=_mesh,
    out_type=jax.ShapeDtypeStruct((N // GRP, GRP, EMBED), jnp.float32),
    scratch_types=[
        pltpu.VMEM((BPW,), jnp.int32),
        pltpu.VMEM((BPW // GRP, GRP, EMBED), jnp.float32),
        pltpu.SemaphoreType.DMA,
    ],
)
def _gather_kernel(idx_hbm, table_hbm, out_hbm, idx_v, rows_v, sem):
    wid = lax.axis_index("s") * NUM_CORES + lax.axis_index("c")
    base = wid * BPW
    pltpu.sync_copy(idx_hbm.at[pl.ds(base, BPW)], idx_v)

    def group(g, _):
        vec = idx_v[pl.ds(g * LANES, LANES)]
        for k in range(LANES):
            row = vec[k]
            j = g * LANES + k
            pltpu.async_copy(
                table_hbm.at[row // GRP].at[pl.ds(row % GRP, 1)],
                rows_v.at[j // GRP].at[pl.ds(j % GRP, 1)],
                sem,
            )
        return 0

    lax.fori_loop(0, NGROUP, group, 0)
    pltpu.make_async_copy(
        table_hbm.at[pl.ds(0, BPW // GRP)], rows_v, sem
    ).wait()
    pltpu.sync_copy(rows_v, out_hbm.at[pl.ds(base // GRP, BPW // GRP)])


def kernel(data, ivectors):
    table3 = ivectors.reshape(VOCAB // GRP, GRP, EMBED)
    out = _gather_kernel(data.astype(jnp.int32), table3)
    return out.reshape(N, EMBED)


# speedup vs baseline: 2.5563x; 1.0013x over previous
import functools

import jax
import jax.numpy as jnp
from jax import lax
from jax.experimental import pallas as pl
from jax.experimental.pallas import tpu as pltpu
from jax.experimental.pallas import tpu_sc as plsc

N = 16384
EMBED = 64
GRP = 8
VOCAB = 1000000
NUM_CORES = 2
NUM_SUBCORES = 16
NW = NUM_CORES * NUM_SUBCORES
BPW = N // NW
LANES = 16
NGROUP = BPW // LANES

_mesh = plsc.VectorSubcoreMesh(core_axis_name="c", subcore_axis_name="s")


@functools.partial(
    pl.kernel,
    mesh=_mesh,
    out_type=jax.ShapeDtypeStruct((N, EMBED), jnp.float32),
    scratch_types=[
        pltpu.VMEM((BPW,), jnp.int32),
        pltpu.VMEM((BPW, EMBED), jnp.float32),
        pltpu.SemaphoreType.DMA,
    ],
)
def _gather_kernel(idx_hbm, table_hbm, out_hbm, idx_v, rows_v, sem):
    wid = lax.axis_index("s") * NUM_CORES + lax.axis_index("c")
    base = wid * BPW
    pltpu.sync_copy(idx_hbm.at[pl.ds(base, BPW)], idx_v)

    def group(g, _):
        vec = idx_v[pl.ds(g * LANES, LANES)]
        for k in range(LANES):
            row = vec[k]
            j = g * LANES + k
            pltpu.async_copy(
                table_hbm.at[row // GRP].at[pl.ds(row % GRP, 1)],
                rows_v.at[pl.ds(j, 1)],
                sem,
            )
        return 0

    lax.fori_loop(0, NGROUP, group, 0)
    pltpu.make_async_copy(
        out_hbm.at[pl.ds(0, BPW)], rows_v, sem
    ).wait()
    pltpu.sync_copy(rows_v, out_hbm.at[pl.ds(base, BPW)])


def kernel(data, ivectors):
    table3 = ivectors.reshape(VOCAB // GRP, GRP, EMBED)
    return _gather_kernel(data.astype(jnp.int32), table3)


# speedup vs baseline: 2.5573x; 1.0004x over previous
import functools

import jax
import jax.numpy as jnp
from jax import lax
from jax.experimental import pallas as pl
from jax.experimental.pallas import tpu as pltpu
from jax.experimental.pallas import tpu_sc as plsc

N = 16384
EMBED = 64
GRP = 8
VOCAB = 1000000
NUM_CORES = 2
NUM_SUBCORES = 16
NW = NUM_CORES * NUM_SUBCORES
BPW = N // NW
HALF = BPW // 2
LANES = 16
NGROUP = HALF // LANES

_mesh = plsc.VectorSubcoreMesh(core_axis_name="c", subcore_axis_name="s")


@functools.partial(
    pl.kernel,
    mesh=_mesh,
    out_type=jax.ShapeDtypeStruct((N, EMBED), jnp.float32),
    scratch_types=[
        pltpu.VMEM((BPW,), jnp.int32),
        pltpu.VMEM((BPW, EMBED), jnp.float32),
        pltpu.SemaphoreType.DMA,
        pltpu.SemaphoreType.DMA,
    ],
)
def _gather_kernel(idx_hbm, table_hbm, out_hbm, idx_v, rows_v, sem0, sem1):
    wid = lax.axis_index("s") * NUM_CORES + lax.axis_index("c")
    base = wid * BPW
    pltpu.sync_copy(idx_hbm.at[pl.ds(base, BPW)], idx_v)

    def make_body(half, sem):
        def group(g, _):
            vec = idx_v[pl.ds(half * HALF + g * LANES, LANES)]
            for k in range(LANES):
                row = vec[k]
                j = half * HALF + g * LANES + k
                pltpu.async_copy(
                    table_hbm.at[row // GRP].at[pl.ds(row % GRP, 1)],
                    rows_v.at[pl.ds(j, 1)],
                    sem,
                )
            return 0

        return group

    lax.fori_loop(0, NGROUP, make_body(0, sem0), 0)
    lax.fori_loop(0, NGROUP, make_body(1, sem1), 0)
    pltpu.make_async_copy(
        out_hbm.at[pl.ds(0, HALF)], rows_v.at[pl.ds(0, HALF)], sem0
    ).wait()
    pltpu.sync_copy(
        rows_v.at[pl.ds(0, HALF)], out_hbm.at[pl.ds(base, HALF)]
    )
    pltpu.make_async_copy(
        out_hbm.at[pl.ds(0, HALF)], rows_v.at[pl.ds(HALF, HALF)], sem1
    ).wait()
    pltpu.sync_copy(
        rows_v.at[pl.ds(HALF, HALF)], out_hbm.at[pl.ds(base + HALF, HALF)]
    )


def kernel(data, ivectors):
    table3 = ivectors.reshape(VOCAB // GRP, GRP, EMBED)
    return _gather_kernel(data.astype(jnp.int32), table3)
